# baseline (speedup 1.0000x reference)
import jax
import jax.numpy as jnp
import numpy as np
from jax import lax
from jax.experimental import pallas as pl
from jax.experimental.pallas import tpu as pltpu

DIM = 1024
HEADS = 8
DIM_HEAD = 128
SCALE = DIM_HEAD ** -0.5
EPS = 1e-10
ROPE_THETA = 10000.0
NEG_INF = -1e30
CLAMP = 115.0
LOG2E = 1.4426950408889634

BQ = 64

_DEF = jax.lax.Precision.DEFAULT


def _rope_tables(n, dtype):
    inv_freq = 1.0 / (ROPE_THETA ** (np.arange(0, DIM_HEAD, 2, dtype=np.float32) / DIM_HEAD))
    freqs = np.arange(n, dtype=np.float32)[:, None] * inv_freq[None, :]
    freqs = np.repeat(freqs, 2, axis=-1)
    sin = np.sin(freqs)
    sin[:, 0::2] *= -1.0
    return jnp.asarray(np.cos(freqs), dtype), jnp.asarray(sin, dtype)


def _qkv_kernel(x_ref, g_ref, w_ref, o_ref):
    x = x_ref[0]
    ss = jnp.sum(x * x, axis=-1, keepdims=True)
    scale = (DIM ** 0.5) * lax.rsqrt(jnp.maximum(ss, 1e-24))
    xn = x * scale * g_ref[...]
    o_ref[0] = lax.dot_general(xn, w_ref[...], (((1,), (1,)), ((), ())),
                               precision=_DEF)


def _rot_half(t, even):
    nxt = pltpu.roll(t, DIM_HEAD - 1, 1)
    prv = pltpu.roll(t, 1, 1)
    return jnp.where(even, nxt, prv)


def _attn_kernel(q_ref, k_ref, v_ref, cos_ref, sin_ref, dbias_ref, mkv_ref,
                 mnr_ref, gate_ref, wout_ref, o_ref, okv_ref, onorm_ref,
                 hacc_ref):
    h = pl.program_id(1)
    n = q_ref.shape[1]
    q = q_ref[0]
    k = k_ref[0]
    v = v_ref[0]
    cos = cos_ref[...]
    sin = sin_ref[...]
    even = lax.broadcasted_iota(jnp.int32, (n, DIM_HEAD), 1) % 2 == 0
    dbias = dbias_ref[...]

    qs = q * (SCALE * LOG2E)
    q_rot = qs * cos + _rot_half(qs, even) * sin
    k_rot = k * cos + _rot_half(k, even) * sin

    mkv = mkv_ref[0, 0]
    mnr = mnr_ref[0, 0]

    qf = jnp.where(q > 0, q + 1.0, jnp.exp(q))
    numer = lax.dot_general(qf, mkv, (((1,), (0,)), ((), ())), precision=_DEF)
    denom = jnp.sum(qf * mnr, axis=-1, keepdims=True)
    mem_out = numer / jnp.maximum(denom, EPS)

    blocks = []
    for i in range(n // BQ):
        lo, hi = i * BQ, (i + 1) * BQ
        qb = q_rot[lo:hi]
        s = lax.dot_general(qb, k_rot[:hi], (((1,), (1,)), ((), ())),
                            precision=_DEF)
        sd = s[:, lo:hi] + dbias
        if i:
            s = jnp.concatenate([s[:, :lo], sd], axis=-1)
        else:
            s = sd
        p = jnp.exp2(jnp.minimum(s, CLAMP))
        l = jnp.sum(p, axis=-1, keepdims=True)
        ob = lax.dot_general(p, v[:hi], (((1,), (0,)), ((), ())),
                             precision=_DEF)
        blocks.append(ob / l)
    attn = jnp.concatenate(blocks, axis=0)

    g = gate_ref[0, 0, 0]
    off = pl.multiple_of(h * DIM_HEAD, DIM_HEAD)
    hacc_ref[:, pl.ds(off, DIM_HEAD)] = attn * g + mem_out * (1.0 - g)

    @pl.when(h == HEADS - 1)
    def _():
        o_ref[0] = lax.dot_general(hacc_ref[...], wout_ref[...],
                                   (((1,), (1,)), ((), ())), precision=_DEF)

    kf = jnp.where(k > 0, k + 1.0, jnp.exp(k))
    dnum = lax.dot_general(kf, mkv, (((1,), (0,)), ((), ())), precision=_DEF)
    dden = jnp.sum(kf * mnr, axis=-1, keepdims=True)
    v_new = v - dnum / jnp.maximum(dden, EPS)
    nkv = lax.dot_general(kf, v_new, (((0,), (0,)), ((), ())), precision=_DEF)
    okv_ref[0, 0] = nkv + mkv
    onorm_ref[0, 0] = jnp.sum(kf, axis=0, keepdims=True) + mnr


def kernel(x, gamma, w_qkv, w_out, head_gates, mem_kv, mem_norm):
    b, n, _ = x.shape
    f32 = jnp.float32

    cos, sin = _rope_tables(n, f32)
    dbias = jnp.asarray(
        np.where(np.arange(BQ)[None, :] > np.arange(BQ)[:, None], NEG_INF, 0.0),
        f32)
    mnorm_row = mem_norm[:, :, None, :]
    gates = jax.nn.sigmoid(head_gates).reshape(HEADS, 1, 1)

    ncb = 3
    cw = 3 * HEADS * DIM_HEAD // ncb
    qkv = pl.pallas_call(
        _qkv_kernel,
        grid=(b, ncb),
        in_specs=[
            pl.BlockSpec((1, n, DIM), lambda i, j: (i, 0, 0)),
            pl.BlockSpec((1, DIM), lambda i, j: (0, 0)),
            pl.BlockSpec((cw, DIM), lambda i, j: (j, 0)),
        ],
        out_specs=pl.BlockSpec((1, n, cw), lambda i, j: (i, 0, j)),
        out_shape=jax.ShapeDtypeStruct((b, n, 3 * HEADS * DIM_HEAD), f32),
        compiler_params=pltpu.CompilerParams(
            dimension_semantics=("parallel", "parallel"),
            vmem_limit_bytes=100 * 1024 * 1024,
        ),
        name="qkv_proj",
    )(x, gamma.reshape(1, DIM), w_qkv)

    d = DIM_HEAD
    out, new_kv, new_norm = pl.pallas_call(
        _attn_kernel,
        grid=(b, HEADS),
        in_specs=[
            pl.BlockSpec((1, n, d), lambda i, j: (i, 0, j)),
            pl.BlockSpec((1, n, d), lambda i, j: (i, 0, HEADS + j)),
            pl.BlockSpec((1, n, d), lambda i, j: (i, 0, 2 * HEADS + j)),
            pl.BlockSpec((n, d), lambda i, j: (0, 0)),
            pl.BlockSpec((n, d), lambda i, j: (0, 0)),
            pl.BlockSpec((BQ, BQ), lambda i, j: (0, 0)),
            pl.BlockSpec((1, 1, d, d), lambda i, j: (i, j, 0, 0)),
            pl.BlockSpec((1, 1, 1, d), lambda i, j: (i, j, 0, 0)),
            pl.BlockSpec((1, 1, 1), lambda i, j: (j, 0, 0)),
            pl.BlockSpec((DIM, HEADS * d), lambda i, j: (0, 0)),
        ],
        out_specs=[
            pl.BlockSpec((1, n, DIM), lambda i, j: (i, 0, 0)),
            pl.BlockSpec((1, 1, d, d), lambda i, j: (i, j, 0, 0)),
            pl.BlockSpec((1, 1, 1, d), lambda i, j: (i, j, 0, 0)),
        ],
        out_shape=[
            jax.ShapeDtypeStruct((b, n, DIM), f32),
            jax.ShapeDtypeStruct((b, HEADS, d, d), f32),
            jax.ShapeDtypeStruct((b, HEADS, 1, d), f32),
        ],
        scratch_shapes=[pltpu.VMEM((n, HEADS * d), f32)],
        compiler_params=pltpu.CompilerParams(
            dimension_semantics=("parallel", "arbitrary"),
            vmem_limit_bytes=100 * 1024 * 1024,
        ),
        name="causal_attn_mem",
    )(qkv, qkv, qkv, cos, sin, dbias, mem_kv, mnorm_row, gates, w_out)

    return out, new_kv, new_norm.reshape(b, HEADS, d)


# speedup vs baseline: 1.4102x; 1.4102x over previous
import jax
import jax.numpy as jnp
import numpy as np
from jax import lax
from jax.experimental import pallas as pl
from jax.experimental.pallas import tpu as pltpu

DIM = 1024
HEADS = 8
DIM_HEAD = 128
SCALE = DIM_HEAD ** -0.5
EPS = 1e-10
ROPE_THETA = 10000.0
NEG_INF = -1e30
CLAMP = 115.0
LOG2E = 1.4426950408889634

BQ = 128

_DEF = jax.lax.Precision.DEFAULT


def _rope_tables(n, dtype):
    inv_freq = 1.0 / (ROPE_THETA ** (np.arange(0, DIM_HEAD, 2, dtype=np.float32) / DIM_HEAD))
    freqs = np.arange(n, dtype=np.float32)[:, None] * inv_freq[None, :]
    freqs = np.repeat(freqs, 2, axis=-1)
    sin = np.sin(freqs)
    sin[:, 0::2] *= -1.0
    return jnp.asarray(np.cos(freqs), dtype), jnp.asarray(sin, dtype)


def _qkv_kernel(x_ref, g_ref, w_ref, o_ref):
    x = x_ref[0]
    ss = jnp.sum(x * x, axis=-1, keepdims=True)
    scale = (DIM ** 0.5) * lax.rsqrt(jnp.maximum(ss, 1e-24))
    xn = x * scale * g_ref[...]
    o_ref[0] = lax.dot_general(xn, w_ref[...], (((1,), (1,)), ((), ())),
                               precision=_DEF)


def _rot_half(t, even):
    nxt = pltpu.roll(t, DIM_HEAD - 1, 1)
    prv = pltpu.roll(t, 1, 1)
    return jnp.where(even, nxt, prv)


def _attn_kernel(q_ref, k_ref, v_ref, cos_ref, sin_ref, dbias_ref, mkv_ref,
                 mnr_ref, gate_ref, wout_ref, o_ref, okv_ref, onorm_ref,
                 hacc_ref):
    h = pl.program_id(1)
    n = q_ref.shape[1]
    q = q_ref[0]
    k = k_ref[0]
    v = v_ref[0]
    cos = cos_ref[...]
    sin = sin_ref[...]
    even = lax.broadcasted_iota(jnp.int32, (n, DIM_HEAD), 1) % 2 == 0
    dbias = dbias_ref[...]

    qs = q * (SCALE * LOG2E)
    q_rot = qs * cos + _rot_half(qs, even) * sin
    k_rot = k * cos + _rot_half(k, even) * sin

    mkv = mkv_ref[0, 0]
    mnr = mnr_ref[0, 0]

    qf = jnp.where(q > 0, q + 1.0, jnp.exp(q))
    numer = lax.dot_general(qf, mkv, (((1,), (0,)), ((), ())), precision=_DEF)
    denom = jnp.sum(qf * mnr, axis=-1, keepdims=True)
    mem_out = numer / jnp.maximum(denom, EPS)

    blocks = []
    for i in range(n // BQ):
        lo, hi = i * BQ, (i + 1) * BQ
        qb = q_rot[lo:hi]
        s = lax.dot_general(qb, k_rot[:hi], (((1,), (1,)), ((), ())),
                            precision=_DEF)
        sd = s[:, lo:hi] + dbias
        if i:
            s = jnp.concatenate([s[:, :lo], sd], axis=-1)
        else:
            s = sd
        p = jnp.exp2(jnp.minimum(s, CLAMP))
        l = jnp.sum(p, axis=-1, keepdims=True)
        ob = lax.dot_general(p, v[:hi], (((1,), (0,)), ((), ())),
                             precision=_DEF)
        blocks.append(ob / l)
    attn = jnp.concatenate(blocks, axis=0)

    g = gate_ref[0, 0, 0]
    off = pl.multiple_of(h * DIM_HEAD, DIM_HEAD)
    hacc_ref[:, pl.ds(off, DIM_HEAD)] = attn * g + mem_out * (1.0 - g)

    @pl.when(h == HEADS - 1)
    def _():
        o_ref[0] = lax.dot_general(hacc_ref[...], wout_ref[...],
                                   (((1,), (1,)), ((), ())), precision=_DEF)

    kf = jnp.where(k > 0, k + 1.0, jnp.exp(k))
    dnum = lax.dot_general(kf, mkv, (((1,), (0,)), ((), ())), precision=_DEF)
    dden = jnp.sum(kf * mnr, axis=-1, keepdims=True)
    v_new = v - dnum / jnp.maximum(dden, EPS)
    nkv = lax.dot_general(kf, v_new, (((0,), (0,)), ((), ())), precision=_DEF)
    okv_ref[0, 0] = nkv + mkv
    onorm_ref[0, 0] = jnp.sum(kf, axis=0, keepdims=True) + mnr


def kernel(x, gamma, w_qkv, w_out, head_gates, mem_kv, mem_norm):
    b, n, _ = x.shape
    f32 = jnp.float32

    cos, sin = _rope_tables(n, f32)
    dbias = jnp.asarray(
        np.where(np.arange(BQ)[None, :] > np.arange(BQ)[:, None], NEG_INF, 0.0),
        f32)
    mnorm_row = mem_norm[:, :, None, :]
    gates = jax.nn.sigmoid(head_gates).reshape(HEADS, 1, 1)

    ncb = 3
    cw = 3 * HEADS * DIM_HEAD // ncb
    qkv = pl.pallas_call(
        _qkv_kernel,
        grid=(b, ncb),
        in_specs=[
            pl.BlockSpec((1, n, DIM), lambda i, j: (i, 0, 0)),
            pl.BlockSpec((1, DIM), lambda i, j: (0, 0)),
            pl.BlockSpec((cw, DIM), lambda i, j: (j, 0)),
        ],
        out_specs=pl.BlockSpec((1, n, cw), lambda i, j: (i, 0, j)),
        out_shape=jax.ShapeDtypeStruct((b, n, 3 * HEADS * DIM_HEAD), f32),
        compiler_params=pltpu.CompilerParams(
            dimension_semantics=("parallel", "parallel"),
            vmem_limit_bytes=100 * 1024 * 1024,
        ),
        name="qkv_proj",
    )(x, gamma.reshape(1, DIM), w_qkv)

    d = DIM_HEAD
    out, new_kv, new_norm = pl.pallas_call(
        _attn_kernel,
        grid=(b, HEADS),
        in_specs=[
            pl.BlockSpec((1, n, d), lambda i, j: (i, 0, j)),
            pl.BlockSpec((1, n, d), lambda i, j: (i, 0, HEADS + j)),
            pl.BlockSpec((1, n, d), lambda i, j: (i, 0, 2 * HEADS + j)),
            pl.BlockSpec((n, d), lambda i, j: (0, 0)),
            pl.BlockSpec((n, d), lambda i, j: (0, 0)),
            pl.BlockSpec((BQ, BQ), lambda i, j: (0, 0)),
            pl.BlockSpec((1, 1, d, d), lambda i, j: (i, j, 0, 0)),
            pl.BlockSpec((1, 1, 1, d), lambda i, j: (i, j, 0, 0)),
            pl.BlockSpec((1, 1, 1), lambda i, j: (j, 0, 0)),
            pl.BlockSpec((DIM, HEADS * d), lambda i, j: (0, 0)),
        ],
        out_specs=[
            pl.BlockSpec((1, n, DIM), lambda i, j: (i, 0, 0)),
            pl.BlockSpec((1, 1, d, d), lambda i, j: (i, j, 0, 0)),
            pl.BlockSpec((1, 1, 1, d), lambda i, j: (i, j, 0, 0)),
        ],
        out_shape=[
            jax.ShapeDtypeStruct((b, n, DIM), f32),
            jax.ShapeDtypeStruct((b, HEADS, d, d), f32),
            jax.ShapeDtypeStruct((b, HEADS, 1, d), f32),
        ],
        scratch_shapes=[pltpu.VMEM((n, HEADS * d), f32)],
        compiler_params=pltpu.CompilerParams(
            dimension_semantics=("parallel", "arbitrary"),
            vmem_limit_bytes=100 * 1024 * 1024,
        ),
        name="causal_attn_mem",
    )(qkv, qkv, qkv, cos, sin, dbias, mem_kv, mnorm_row, gates, w_out)

    return out, new_kv, new_norm.reshape(b, HEADS, d)
